# baseline (speedup 1.0000x reference)
import functools

import jax
import jax.numpy as jnp
from jax import lax
from jax.experimental import pallas as pl
from jax.experimental.pallas import tpu as pltpu
from jax.experimental.pallas import tpu_sc as plsc

_D = 128
_B = 16384
_NW = 16
_L = 16
_BPW = _B // _NW
_CH = 128
_NCH = _BPW // _CH
_HALF = _NCH // 2

_mesh = plsc.VectorSubcoreMesh(
    core_axis_name="c", subcore_axis_name="s", num_cores=1)


@functools.partial(
    pl.kernel,
    mesh=_mesh,
    out_type=jax.ShapeDtypeStruct((_B,), jnp.float32),
    scratch_types=[
        pltpu.VMEM((_BPW,), jnp.int32),
        pltpu.VMEM((_BPW,), jnp.int32),
        pltpu.VMEM((_NCH, _CH), jnp.int32),
        pltpu.VMEM((_BPW,), jnp.float32),
        pltpu.SemaphoreType.DMA,
        pltpu.SemaphoreType.DMA,
        pltpu.SemaphoreType.DMA,
        pltpu.SemaphoreType.DMA,
    ],
)
def _gather_kernel(uid_hbm, iid_hbm, tab_hbm, out_hbm,
                   uid_v, iid_v, idx_v, val_v,
                   sem_in, sem_g1, sem_g2, sem_st):
    wid = lax.axis_index("s")
    base = wid * _BPW
    ld_u = pltpu.async_copy(uid_hbm.at[pl.ds(base, _BPW)], uid_v, sem_in)
    ld_i = pltpu.async_copy(iid_hbm.at[pl.ds(base, _BPW)], iid_v, sem_in)
    ld_u.wait()
    ld_i.wait()
    gathers = []
    for j in range(_NCH):
        for k in range(_CH // _L):
            off = j * _CH + k * _L
            u = uid_v[pl.ds(off, _L)]
            it = iid_v[pl.ds(off, _L)]
            idx_v[j, pl.ds(k * _L, _L)] = u * _D + it
        sem_g = sem_g1 if j < _HALF else sem_g2
        gathers.append(
            pltpu.async_copy(tab_hbm.at[idx_v.at[j]],
                             val_v.at[pl.ds(j * _CH, _CH)], sem_g))
    half = _HALF * _CH
    for g in gathers[:_HALF]:
        g.wait()
    st1 = pltpu.async_copy(val_v.at[pl.ds(0, half)],
                           out_hbm.at[pl.ds(base, half)], sem_st)
    for g in gathers[_HALF:]:
        g.wait()
    st2 = pltpu.async_copy(val_v.at[pl.ds(half, half)],
                           out_hbm.at[pl.ds(base + half, half)], sem_st)
    st1.wait()
    st2.wait()


def kernel(user_ids, item_ids, items_cnts):
    flat_table = items_cnts.reshape(-1)
    return _gather_kernel(user_ids.astype(jnp.int32),
                          item_ids.astype(jnp.int32),
                          flat_table)


# speedup vs baseline: 1.0053x; 1.0053x over previous
import functools

import jax
import jax.numpy as jnp
from jax import lax
from jax.experimental import pallas as pl
from jax.experimental.pallas import tpu as pltpu
from jax.experimental.pallas import tpu_sc as plsc

_D = 128
_B = 16384
_NW = 16
_L = 16
_BPW = _B // _NW
_CH = 128
_NCH = _BPW // _CH
_HALF = _NCH // 2

_mesh = plsc.VectorSubcoreMesh(
    core_axis_name="c", subcore_axis_name="s", num_cores=1)


@functools.partial(
    pl.kernel,
    mesh=_mesh,
    out_type=jax.ShapeDtypeStruct((_B,), jnp.float32),
    scratch_types=[
        pltpu.VMEM((_BPW,), jnp.int32),
        pltpu.VMEM((_BPW,), jnp.int32),
        pltpu.VMEM((_NCH, _CH), jnp.int32),
        pltpu.VMEM((_BPW,), jnp.float32),
        pltpu.SemaphoreType.DMA,
        pltpu.SemaphoreType.DMA,
        pltpu.SemaphoreType.DMA,
        pltpu.SemaphoreType.DMA,
    ],
)
def _gather_kernel(uid_hbm, iid_hbm, tab_hbm, out_hbm,
                   uid_v, iid_v, idx_v, val_v,
                   sem_in, sem_g1, sem_g2, sem_st):
    wid = lax.axis_index("s")
    base = wid * _BPW
    hw = _BPW // 2
    ld_u1 = pltpu.async_copy(uid_hbm.at[pl.ds(base, hw)],
                             uid_v.at[pl.ds(0, hw)], sem_in)
    ld_i1 = pltpu.async_copy(iid_hbm.at[pl.ds(base, hw)],
                             iid_v.at[pl.ds(0, hw)], sem_in)
    ld_u2 = pltpu.async_copy(uid_hbm.at[pl.ds(base + hw, hw)],
                             uid_v.at[pl.ds(hw, hw)], sem_in)
    ld_i2 = pltpu.async_copy(iid_hbm.at[pl.ds(base + hw, hw)],
                             iid_v.at[pl.ds(hw, hw)], sem_in)
    ld_u1.wait()
    ld_i1.wait()
    gathers = []
    for j in range(_NCH):
        if j == _HALF:
            ld_u2.wait()
            ld_i2.wait()
        for k in range(_CH // _L):
            off = j * _CH + k * _L
            u = uid_v[pl.ds(off, _L)]
            it = iid_v[pl.ds(off, _L)]
            idx_v[j, pl.ds(k * _L, _L)] = u * _D + it
        sem_g = sem_g1 if j < _HALF else sem_g2
        gathers.append(
            pltpu.async_copy(tab_hbm.at[idx_v.at[j]],
                             val_v.at[pl.ds(j * _CH, _CH)], sem_g))
    half = _HALF * _CH
    for g in gathers[:_HALF]:
        g.wait()
    st1 = pltpu.async_copy(val_v.at[pl.ds(0, half)],
                           out_hbm.at[pl.ds(base, half)], sem_st)
    for g in gathers[_HALF:]:
        g.wait()
    st2 = pltpu.async_copy(val_v.at[pl.ds(half, half)],
                           out_hbm.at[pl.ds(base + half, half)], sem_st)
    st1.wait()
    st2.wait()


def kernel(user_ids, item_ids, items_cnts):
    flat_table = items_cnts.reshape(-1)
    return _gather_kernel(user_ids.astype(jnp.int32),
                          item_ids.astype(jnp.int32),
                          flat_table)
